# baseline (speedup 1.0000x reference)
import functools

import jax
import jax.numpy as jnp
from jax import lax
from jax.experimental import pallas as pl
from jax.experimental.pallas import tpu as pltpu
from jax.experimental.pallas import tpu_sc as plsc

NUM_EXPERTS = 64
TOPK = 2
TOKENS = 16384
HIDDEN = 2048
BLOCK_T = 2048

_INFO = plsc.get_sparse_core_info()
_NC = _INFO.num_cores
_NS = _INFO.num_subcores
_NW = _NC * _NS
_L = _INFO.num_lanes
TOK_PER_W = TOKENS // _NW


def _matmul_body(x_ref, w_ref, lt_ref):
    lt_ref[...] = lax.dot_general(
        w_ref[...], x_ref[...],
        dimension_numbers=(((1,), (1,)), ((), ())),
        preferred_element_type=jnp.float32,
    )


def _tc_logits_t(x, W):
    n_tok = x.shape[0]
    return pl.pallas_call(
        _matmul_body,
        grid=(n_tok // BLOCK_T,),
        in_specs=[
            pl.BlockSpec((BLOCK_T, HIDDEN), lambda i: (i, 0)),
            pl.BlockSpec((NUM_EXPERTS, HIDDEN), lambda i: (0, 0)),
        ],
        out_specs=pl.BlockSpec((NUM_EXPERTS, BLOCK_T), lambda i: (0, i)),
        out_shape=jax.ShapeDtypeStruct((NUM_EXPERTS, n_tok), jnp.float32),
    )(x, W)


_HALF = TOK_PER_W // 2


def _sc_router_body(lt_hbm, s_hbm, i_hbm, lg0_v, lg1_v, s_v, i_v, sem0, sem1):
    wid = lax.axis_index("s") * _NC + lax.axis_index("c")
    base = wid * TOK_PER_W
    cp0 = pltpu.async_copy(lt_hbm.at[:, pl.ds(base, _HALF)], lg0_v, sem0)
    cp1 = pltpu.async_copy(lt_hbm.at[:, pl.ds(base + _HALF, _HALF)], lg1_v, sem1)

    def make_group(lg_v, off):
        def _group(g):
            col = pl.ds(g * _L, _L)
            ocol = pl.ds(off + g * _L, _L)
            m1 = lg_v[0, col]
            i1 = jnp.zeros((_L,), jnp.int32)
            m2 = jnp.full((_L,), -jnp.inf, jnp.float32)
            i2 = jnp.zeros((_L,), jnp.int32)
            for e in range(1, NUM_EXPERTS):
                v = lg_v[e, col]
                ev = jnp.full((_L,), e, jnp.int32)
                gt1 = v > m1
                gt2 = v > m2
                i2 = jnp.where(gt1, i1, jnp.where(gt2, ev, i2))
                m2 = jnp.where(gt1, m1, jnp.where(gt2, v, m2))
                i1 = jnp.where(gt1, ev, i1)
                m1 = jnp.where(gt1, v, m1)
            e2 = jnp.exp(m2 - m1)
            rden = 1.0 / (1.0 + e2)
            s_v[0, ocol] = rden
            s_v[1, ocol] = e2 * rden
            i_v[0, ocol] = i1
            i_v[1, ocol] = i2
        return _group

    cp0.wait()
    plsc.parallel_loop(0, _HALF // _L, 1, unroll=4)(make_group(lg0_v, 0))
    cp1.wait()
    plsc.parallel_loop(0, _HALF // _L, 1, unroll=4)(make_group(lg1_v, _HALF))

    pltpu.sync_copy(s_v, s_hbm.at[:, pl.ds(base, TOK_PER_W)])
    pltpu.sync_copy(i_v, i_hbm.at[:, pl.ds(base, TOK_PER_W)])


def _sc_router(lt):
    n_tok = lt.shape[1]
    run = pl.kernel(
        _sc_router_body,
        mesh=plsc.VectorSubcoreMesh(core_axis_name="c", subcore_axis_name="s"),
        out_type=[
            jax.ShapeDtypeStruct((TOPK, n_tok), jnp.float32),
            jax.ShapeDtypeStruct((TOPK, n_tok), jnp.int32),
        ],
        scratch_types=[
            pltpu.VMEM((NUM_EXPERTS, _HALF), jnp.float32),
            pltpu.VMEM((NUM_EXPERTS, _HALF), jnp.float32),
            pltpu.VMEM((TOPK, TOK_PER_W), jnp.float32),
            pltpu.VMEM((TOPK, TOK_PER_W), jnp.int32),
            pltpu.SemaphoreType.DMA,
            pltpu.SemaphoreType.DMA,
        ],
    )
    return run(lt)


@jax.jit
def kernel(input, W):
    lt = _tc_logits_t(input, W)
    s, i = _sc_router(lt)
    return s.T, i.T


# speedup vs baseline: 1.0051x; 1.0051x over previous
import jax
import jax.numpy as jnp
from jax import lax
from jax.experimental import pallas as pl
from jax.experimental.pallas import tpu as pltpu
from jax.experimental.pallas import tpu_sc as plsc

NUM_EXPERTS = 64
TOPK = 2
TOKENS = 16384
HIDDEN = 2048
BLOCK_T = 2048

_INFO = plsc.get_sparse_core_info()
_NC = _INFO.num_cores
_NS = _INFO.num_subcores
_NW = _NC * _NS
_L = _INFO.num_lanes
TOK_PER_W = TOKENS // _NW


def _matmul_body(x_ref, w_ref, lt_ref):
    lt_ref[...] = lax.dot_general(
        w_ref[...], x_ref[...],
        dimension_numbers=(((1,), (1,)), ((), ())),
        preferred_element_type=jnp.float32,
    )


def _tc_logits_t(x, W):
    n_tok = x.shape[0]
    return pl.pallas_call(
        _matmul_body,
        grid=(n_tok // BLOCK_T,),
        in_specs=[
            pl.BlockSpec((BLOCK_T, HIDDEN), lambda i: (i, 0)),
            pl.BlockSpec((NUM_EXPERTS, HIDDEN), lambda i: (0, 0)),
        ],
        out_specs=pl.BlockSpec((NUM_EXPERTS, BLOCK_T), lambda i: (0, i)),
        out_shape=jax.ShapeDtypeStruct((NUM_EXPERTS, n_tok), jnp.float32),
    )(x, W)


_HALF = TOK_PER_W // 2


def _sc_router_body(lt_hbm, s_hbm, i_hbm, lg0_v, lg1_v, s_v, i_v, sem0, sem1):
    wid = lax.axis_index("s") * _NC + lax.axis_index("c")
    base = wid * TOK_PER_W
    cp0 = pltpu.async_copy(lt_hbm.at[:, pl.ds(base, _HALF)], lg0_v, sem0)
    cp1 = pltpu.async_copy(lt_hbm.at[:, pl.ds(base + _HALF, _HALF)], lg1_v, sem1)

    def make_group(lg_v, off):
        def _group(g):
            col = pl.ds(g * _L, _L)
            ocol = pl.ds(off + g * _L, _L)
            m1 = lg_v[0, col]
            i1 = jnp.zeros((_L,), jnp.int32)
            m2 = jnp.full((_L,), -jnp.inf, jnp.float32)
            i2 = jnp.zeros((_L,), jnp.int32)
            for e in range(1, NUM_EXPERTS):
                v = lg_v[e, col]
                ev = jnp.full((_L,), e, jnp.int32)
                gt1 = v > m1
                gt2 = v > m2
                i2 = jnp.where(gt1, i1, jnp.where(gt2, ev, i2))
                m2 = jnp.where(gt1, m1, jnp.where(gt2, v, m2))
                i1 = jnp.where(gt1, ev, i1)
                m1 = jnp.where(gt1, v, m1)
            e2 = jnp.exp(m2 - m1)
            rden = 1.0 / (1.0 + e2)
            s_v[0, ocol] = rden
            s_v[1, ocol] = e2 * rden
            i_v[0, ocol] = i1
            i_v[1, ocol] = i2
        return _group

    cp0.wait()
    plsc.parallel_loop(0, _HALF // _L, 1, unroll=2)(make_group(lg0_v, 0))
    cp1.wait()
    plsc.parallel_loop(0, _HALF // _L, 1, unroll=2)(make_group(lg1_v, _HALF))

    pltpu.sync_copy(s_v, s_hbm.at[:, pl.ds(base, TOK_PER_W)])
    pltpu.sync_copy(i_v, i_hbm.at[:, pl.ds(base, TOK_PER_W)])


def _sc_router(lt):
    n_tok = lt.shape[1]
    run = pl.kernel(
        _sc_router_body,
        mesh=plsc.VectorSubcoreMesh(core_axis_name="c", subcore_axis_name="s"),
        out_type=[
            jax.ShapeDtypeStruct((TOPK, n_tok), jnp.float32),
            jax.ShapeDtypeStruct((TOPK, n_tok), jnp.int32),
        ],
        scratch_types=[
            pltpu.VMEM((NUM_EXPERTS, _HALF), jnp.float32),
            pltpu.VMEM((NUM_EXPERTS, _HALF), jnp.float32),
            pltpu.VMEM((TOPK, TOK_PER_W), jnp.float32),
            pltpu.VMEM((TOPK, TOK_PER_W), jnp.int32),
            pltpu.SemaphoreType.DMA,
            pltpu.SemaphoreType.DMA,
        ],
    )
    return run(lt)


@jax.jit
def kernel(input, W):
    lt = _tc_logits_t(input, W)
    s, i = _sc_router(lt)
    return s.T, i.T


# speedup vs baseline: 1.0261x; 1.0209x over previous
import jax
import jax.numpy as jnp
from jax import lax
from jax.experimental import pallas as pl
from jax.experimental.pallas import tpu as pltpu
from jax.experimental.pallas import tpu_sc as plsc

NUM_EXPERTS = 64
TOPK = 2
TOKENS = 16384
HIDDEN = 2048
BLOCK_T = 1024

_INFO = plsc.get_sparse_core_info()
_NC = _INFO.num_cores
_NS = _INFO.num_subcores
_NW = _NC * _NS
_L = _INFO.num_lanes
TOK_PER_W = TOKENS // _NW


def _matmul_body(x_ref, w_ref, lt_ref):
    lt_ref[...] = lax.dot_general(
        w_ref[...], x_ref[...],
        dimension_numbers=(((1,), (1,)), ((), ())),
        preferred_element_type=jnp.float32,
    )


def _tc_logits_t(x, W):
    n_tok = x.shape[0]
    return pl.pallas_call(
        _matmul_body,
        grid=(n_tok // BLOCK_T,),
        in_specs=[
            pl.BlockSpec((BLOCK_T, HIDDEN), lambda i: (i, 0)),
            pl.BlockSpec((NUM_EXPERTS, HIDDEN), lambda i: (0, 0)),
        ],
        out_specs=pl.BlockSpec((NUM_EXPERTS, BLOCK_T), lambda i: (0, i)),
        out_shape=jax.ShapeDtypeStruct((NUM_EXPERTS, n_tok), jnp.float32),
    )(x, W)


_HALF = TOK_PER_W // 2


def _sc_router_body(lt_hbm, s_hbm, i_hbm, lg0_v, lg1_v, s_v, i_v, sem0, sem1):
    wid = lax.axis_index("s") * _NC + lax.axis_index("c")
    base = wid * TOK_PER_W
    cp0 = pltpu.async_copy(lt_hbm.at[:, pl.ds(base, _HALF)], lg0_v, sem0)
    cp1 = pltpu.async_copy(lt_hbm.at[:, pl.ds(base + _HALF, _HALF)], lg1_v, sem1)

    def make_group(lg_v, off):
        def _group(g):
            col = pl.ds(g * _L, _L)
            ocol = pl.ds(off + g * _L, _L)
            m1 = lg_v[0, col]
            i1 = jnp.zeros((_L,), jnp.int32)
            m2 = jnp.full((_L,), -jnp.inf, jnp.float32)
            i2 = jnp.zeros((_L,), jnp.int32)
            for e in range(1, NUM_EXPERTS):
                v = lg_v[e, col]
                ev = jnp.full((_L,), e, jnp.int32)
                gt1 = v > m1
                gt2 = v > m2
                i2 = jnp.where(gt1, i1, jnp.where(gt2, ev, i2))
                m2 = jnp.where(gt1, m1, jnp.where(gt2, v, m2))
                i1 = jnp.where(gt1, ev, i1)
                m1 = jnp.where(gt1, v, m1)
            e2 = jnp.exp(m2 - m1)
            rden = 1.0 / (1.0 + e2)
            s_v[0, ocol] = rden
            s_v[1, ocol] = e2 * rden
            i_v[0, ocol] = i1
            i_v[1, ocol] = i2
        return _group

    cp0.wait()
    plsc.parallel_loop(0, _HALF // _L, 1, unroll=2)(make_group(lg0_v, 0))
    cp1.wait()
    plsc.parallel_loop(0, _HALF // _L, 1, unroll=2)(make_group(lg1_v, _HALF))

    pltpu.sync_copy(s_v, s_hbm.at[:, pl.ds(base, TOK_PER_W)])
    pltpu.sync_copy(i_v, i_hbm.at[:, pl.ds(base, TOK_PER_W)])


def _sc_router(lt):
    n_tok = lt.shape[1]
    run = pl.kernel(
        _sc_router_body,
        mesh=plsc.VectorSubcoreMesh(core_axis_name="c", subcore_axis_name="s"),
        out_type=[
            jax.ShapeDtypeStruct((TOPK, n_tok), jnp.float32),
            jax.ShapeDtypeStruct((TOPK, n_tok), jnp.int32),
        ],
        scratch_types=[
            pltpu.VMEM((NUM_EXPERTS, _HALF), jnp.float32),
            pltpu.VMEM((NUM_EXPERTS, _HALF), jnp.float32),
            pltpu.VMEM((TOPK, TOK_PER_W), jnp.float32),
            pltpu.VMEM((TOPK, TOK_PER_W), jnp.int32),
            pltpu.SemaphoreType.DMA,
            pltpu.SemaphoreType.DMA,
        ],
    )
    return run(lt)


@jax.jit
def kernel(input, W):
    lt = _tc_logits_t(input, W)
    s, i = _sc_router(lt)
    return s.T, i.T
